# Initial kernel scaffold; baseline (speedup 1.0000x reference)
import jax
import jax.numpy as jnp
from jax.experimental import pallas as pl


def kernel(data, ivectors, ovectors):
    raise NotImplementedError("write your pallas kernel here")


# speedup vs baseline: 1.8777x; 1.8777x over previous
import functools

import jax
import jax.numpy as jnp
from jax import lax
from jax.experimental import pallas as pl
from jax.experimental.pallas import tpu as pltpu
from jax.experimental.pallas import tpu_sc as plsc

VOCAB = 1000000
EMBED = 64
BATCH = 16384
HIST = 50

NW = 32
TOTAL = BATCH * HIST
R_PER_W = TOTAL // NW
C = 128
NCHUNK = R_PER_W // C
K = 8
NGROUP = NCHUNK // K


@functools.partial(
    pl.kernel,
    mesh=plsc.VectorSubcoreMesh(core_axis_name="c", subcore_axis_name="s"),
    out_type=jax.ShapeDtypeStruct((TOTAL, EMBED), jnp.float32),
    compiler_params=pltpu.CompilerParams(use_tc_tiling_on_sc=False),
    scratch_types=[
        pltpu.VMEM((NCHUNK, C), jnp.int32),
        pltpu.VMEM((K, C, EMBED), jnp.float32),
        pltpu.SemaphoreType.DMA,
        pltpu.SemaphoreType.DMA,
    ],
)
def _gather_rows(idx_hbm, table_hbm, out_hbm, idx_v, rows_v, gsem, ssem):
    cid = lax.axis_index("c")
    sid = lax.axis_index("s")
    wid = sid * 2 + cid
    pltpu.sync_copy(idx_hbm.at[wid], idx_v)
    base = wid * R_PER_W

    def group(g, carry):
        j0 = g * K
        gathers = []
        for b in range(K):
            gathers.append(
                pltpu.async_copy(table_hbm.at[idx_v.at[j0 + b]], rows_v.at[b], gsem)
            )
        stores = []
        for b in range(K):
            gathers[b].wait()
            stores.append(
                pltpu.async_copy(
                    rows_v.at[b], out_hbm.at[pl.ds(base + (j0 + b) * C, C)], ssem
                )
            )
        for b in range(K):
            stores[b].wait()
        return carry

    lax.fori_loop(0, NGROUP, group, 0)


def kernel(data, ivectors, ovectors):
    idx = data.reshape(TOTAL).astype(jnp.int32).reshape(NW, NCHUNK, C)
    flat = _gather_rows(idx, ivectors)
    return flat.reshape(BATCH, HIST, EMBED)
